# Initial kernel scaffold; baseline (speedup 1.0000x reference)
import jax
import jax.numpy as jnp
from jax.experimental import pallas as pl


def kernel(x, fc1_w, fc1_b, bn1_g, bn1_b, eps1, gin1_w1, gin1_b1, gin1_w2, gin1_b2, att_w1, att_b1, att_w2, att_b2, eps2, gin2_w1, gin2_b1, gin2_w2, gin2_b2, ffn_w1, ffn_b1, bn2_g, bn2_b, ffn_w2, ffn_b2, bn3_g, bn3_b):
    raise NotImplementedError("write your pallas kernel here")


# speedup vs baseline: 6.5656x; 6.5656x over previous
import functools

import jax
import jax.numpy as jnp
from jax.experimental import pallas as pl
from jax.experimental.pallas import tpu as pltpu

B, C, H, W = 32, 384, 16, 16
K = 16
HID = 4 * C
Npix = H * W
Ntot = B * Npix
ROWS_BLK = 1024
N_BLKS = Ntot // ROWS_BLK


def _matmul(a, b):
    return jax.lax.dot_general(a, b, (((1,), (0,)), ((), ())),
                               preferred_element_type=jnp.float32)


def _matmul_t(a, b):
    return jax.lax.dot_general(a, b, (((1,), (1,)), ((), ())),
                               preferred_element_type=jnp.float32)


def _matmul_hi(a, b):
    return jax.lax.dot_general(a, b, (((1,), (0,)), ((), ())),
                               preferred_element_type=jnp.float32,
                               precision=jax.lax.Precision.HIGHEST)


def _accum_stats(i, y, stats_ref, shift_ref):
    @pl.when(i == 0)
    def _():
        shift_ref[...] = jnp.mean(y, axis=0, keepdims=True)
        stats_ref[...] = jnp.zeros_like(stats_ref)

    a = shift_ref[...]
    yc = y - a
    stats_ref[...] += jnp.concatenate(
        [jnp.sum(yc, axis=0, keepdims=True),
         jnp.sum(yc * yc, axis=0, keepdims=True)], axis=0)

    @pl.when(i == pl.num_programs(0) - 1)
    def _():
        n = jnp.float32(Ntot)
        d = stats_ref[0:1, :] / n
        var = stats_ref[1:2, :] / n - d * d
        stats_ref[...] = jnp.concatenate([shift_ref[...] + d, var], axis=0)


def _prep_body(x_ref, w_ref, b_ref, y_ref, stats_ref, shift_ref):
    i = pl.program_id(0)
    y = _matmul(x_ref[...], w_ref[...]) + b_ref[...]
    y_ref[...] = y
    _accum_stats(i, y, stats_ref, shift_ref)


def _prep(x_rows, fc1_w, fc1_b):
    return pl.pallas_call(
        _prep_body,
        grid=(N_BLKS,),
        in_specs=[
            pl.BlockSpec((ROWS_BLK, C), lambda i: (i, 0)),
            pl.BlockSpec((C, C), lambda i: (0, 0)),
            pl.BlockSpec((1, C), lambda i: (0, 0)),
        ],
        out_specs=[
            pl.BlockSpec((ROWS_BLK, C), lambda i: (i, 0)),
            pl.BlockSpec((2, C), lambda i: (0, 0)),
        ],
        out_shape=[
            jax.ShapeDtypeStruct((Ntot, C), jnp.float32),
            jax.ShapeDtypeStruct((2, C), jnp.float32),
        ],
        scratch_shapes=[pltpu.VMEM((1, C), jnp.float32)],
    )(x_rows, fc1_w, fc1_b)


def _gnn_body(y0_ref, stats_ref, bn1g_ref, bn1b_ref, eps_ref,
              g1w1_ref, g1b1_ref, g1w2_ref, g1b2_ref,
              w1a_ref, w1b_ref, attb1_ref, w2_ref, attb2_ref,
              g2w1_ref, g2b1_ref, g2w2_ref, g2b2_ref,
              att_ref, emb_ref):
    mean = stats_ref[0:1, :]
    var = stats_ref[1:2, :]
    scale = bn1g_ref[...] / jnp.sqrt(var + 1e-5)
    shift = bn1b_ref[...] - mean * scale
    xn = y0_ref[...] * scale + shift

    rn = jnp.sqrt(jnp.sum(xn * xn, axis=1, keepdims=True))
    nodes = xn / jnp.maximum(rn, 1e-12)

    sq = jnp.sum(nodes * nodes, axis=1, keepdims=True)
    g = _matmul_t(nodes, nodes)
    rowio = jax.lax.broadcasted_iota(jnp.int32, (Npix, Npix), 0)
    colio = jax.lax.broadcasted_iota(jnp.int32, (Npix, Npix), 1)
    d2 = sq + jnp.transpose(sq) - 2.0 * g
    d2 = d2 + jnp.where(rowio == colio, 1e10, 0.0)
    score = -d2

    js = []
    adj = jnp.zeros((Npix, Npix), jnp.float32)
    for _ in range(K):
        m = jnp.max(score, axis=1, keepdims=True)
        cand = jnp.where(score == m, colio, Npix + 1)
        j = jnp.min(cand, axis=1, keepdims=True)
        onehot = (colio == j)
        js.append(j)
        adj = adj + onehot.astype(jnp.float32)
        score = jnp.where(onehot, -1e30, score)

    eps1 = eps_ref[0, 0]
    eps2 = eps_ref[0, 1]
    agg1 = _matmul_hi(adj, nodes)
    h = (1.0 + eps1) * nodes + agg1
    h = _matmul(jnp.maximum(_matmul(h, g1w1_ref[...]) + g1b1_ref[...], 0.0),
                g1w2_ref[...]) + g1b2_ref[...]
    h = jnp.maximum(h, 0.0)

    a_feat = _matmul(h, w1a_ref[...])
    b_feat = _matmul(h, w1b_ref[...]) + attb1_ref[...]
    w2row = w2_ref[...]
    b2 = attb2_ref[0, 0]

    watt = jnp.zeros((Npix, Npix), jnp.float32)
    att_cols = []
    for k in range(K):
        onehot = (colio == js[k]).astype(jnp.float32)
        a_gath = _matmul_hi(onehot, a_feat)
        t = jnp.maximum(a_gath + b_feat, 0.0)
        logit = jnp.sum(t * w2row, axis=1, keepdims=True) + b2
        att_k = jax.nn.sigmoid(logit)
        att_cols.append(att_k)
        watt = watt + att_k * onehot
    att_ref[...] = jnp.concatenate(att_cols, axis=1)

    agg2 = _matmul_hi(watt, h)
    emb = (1.0 + eps2) * h + agg2
    emb = _matmul(jnp.maximum(_matmul(emb, g2w1_ref[...]) + g2b1_ref[...], 0.0),
                  g2w2_ref[...]) + g2b2_ref[...]
    emb_ref[...] = emb


def _gnn(y0, stats, bn1_g, bn1_b, eps12, g1w1, g1b1, g1w2, g1b2,
         w1a, w1b, att_b1, w2row, att_b2, g2w1, g2b1, g2w2, g2b2):
    full = lambda r, c: pl.BlockSpec((r, c), lambda i: (0, 0))
    return pl.pallas_call(
        _gnn_body,
        grid=(B,),
        in_specs=[
            pl.BlockSpec((Npix, C), lambda i: (i, 0)),
            full(2, C), full(1, C), full(1, C), full(1, 2),
            full(C, C), full(1, C), full(C, C), full(1, C),
            full(C, C), full(C, C), full(1, C), full(1, C), full(1, 1),
            full(C, C), full(1, C), full(C, C), full(1, C),
        ],
        out_specs=[
            pl.BlockSpec((Npix, K), lambda i: (i, 0)),
            pl.BlockSpec((Npix, C), lambda i: (i, 0)),
        ],
        out_shape=[
            jax.ShapeDtypeStruct((Ntot, K), jnp.float32),
            jax.ShapeDtypeStruct((Ntot, C), jnp.float32),
        ],
    )(y0, stats, bn1_g, bn1_b, eps12, g1w1, g1b1, g1w2, g1b2,
      w1a, w1b, att_b1, w2row, att_b2, g2w1, g2b1, g2w2, g2b2)


def _ffn1_body(emb_ref, w_ref, b_ref, t_ref, stats_ref, shift_ref):
    i = pl.program_id(0)
    t = _matmul(emb_ref[...], w_ref[...]) + b_ref[...]
    t_ref[...] = t
    _accum_stats(i, t, stats_ref, shift_ref)


def _ffn1(emb, ffn_w1, ffn_b1):
    return pl.pallas_call(
        _ffn1_body,
        grid=(N_BLKS,),
        in_specs=[
            pl.BlockSpec((ROWS_BLK, C), lambda i: (i, 0)),
            pl.BlockSpec((C, HID), lambda i: (0, 0)),
            pl.BlockSpec((1, HID), lambda i: (0, 0)),
        ],
        out_specs=[
            pl.BlockSpec((ROWS_BLK, HID), lambda i: (i, 0)),
            pl.BlockSpec((2, HID), lambda i: (0, 0)),
        ],
        out_shape=[
            jax.ShapeDtypeStruct((Ntot, HID), jnp.float32),
            jax.ShapeDtypeStruct((2, HID), jnp.float32),
        ],
        scratch_shapes=[pltpu.VMEM((1, HID), jnp.float32)],
    )(emb, ffn_w1, ffn_b1)


def _ffn2_body(t_ref, stats_ref, g_ref, b_ref, w_ref, wb_ref, v_ref,
               stats2_ref, shift_ref):
    i = pl.program_id(0)
    mean = stats_ref[0:1, :]
    var = stats_ref[1:2, :]
    scale = g_ref[...] / jnp.sqrt(var + 1e-5)
    shift = b_ref[...] - mean * scale
    u = jax.nn.gelu(t_ref[...] * scale + shift, approximate=True)
    v = _matmul(u, w_ref[...]) + wb_ref[...]
    v_ref[...] = v
    _accum_stats(i, v, stats2_ref, shift_ref)


def _ffn2(t1, stats2, bn2_g, bn2_b, ffn_w2, ffn_b2):
    return pl.pallas_call(
        _ffn2_body,
        grid=(N_BLKS,),
        in_specs=[
            pl.BlockSpec((ROWS_BLK, HID), lambda i: (i, 0)),
            pl.BlockSpec((2, HID), lambda i: (0, 0)),
            pl.BlockSpec((1, HID), lambda i: (0, 0)),
            pl.BlockSpec((1, HID), lambda i: (0, 0)),
            pl.BlockSpec((HID, C), lambda i: (0, 0)),
            pl.BlockSpec((1, C), lambda i: (0, 0)),
        ],
        out_specs=[
            pl.BlockSpec((ROWS_BLK, C), lambda i: (i, 0)),
            pl.BlockSpec((2, C), lambda i: (0, 0)),
        ],
        out_shape=[
            jax.ShapeDtypeStruct((Ntot, C), jnp.float32),
            jax.ShapeDtypeStruct((2, C), jnp.float32),
        ],
        scratch_shapes=[pltpu.VMEM((1, C), jnp.float32)],
    )(t1, stats2, bn2_g, bn2_b, ffn_w2, ffn_b2)


def _ffn3_body(emb_ref, v_ref, stats_ref, g_ref, b_ref, o_ref):
    mean = stats_ref[0:1, :]
    var = stats_ref[1:2, :]
    scale = g_ref[...] / jnp.sqrt(var + 1e-5)
    shift = b_ref[...] - mean * scale
    o_ref[...] = emb_ref[...] + (v_ref[...] * scale + shift)


def _ffn3(emb, v, stats3, bn3_g, bn3_b):
    return pl.pallas_call(
        _ffn3_body,
        grid=(N_BLKS,),
        in_specs=[
            pl.BlockSpec((ROWS_BLK, C), lambda i: (i, 0)),
            pl.BlockSpec((ROWS_BLK, C), lambda i: (i, 0)),
            pl.BlockSpec((2, C), lambda i: (0, 0)),
            pl.BlockSpec((1, C), lambda i: (0, 0)),
            pl.BlockSpec((1, C), lambda i: (0, 0)),
        ],
        out_specs=pl.BlockSpec((ROWS_BLK, C), lambda i: (i, 0)),
        out_shape=jax.ShapeDtypeStruct((Ntot, C), jnp.float32),
    )(emb, v, stats3, bn3_g, bn3_b)


@jax.jit
def kernel(x, fc1_w, fc1_b, bn1_g, bn1_b, eps1, gin1_w1, gin1_b1, gin1_w2,
           gin1_b2, att_w1, att_b1, att_w2, att_b2, eps2, gin2_w1, gin2_b1,
           gin2_w2, gin2_b2, ffn_w1, ffn_b1, bn2_g, bn2_b, ffn_w2, ffn_b2,
           bn3_g, bn3_b):
    x_rows = x.transpose(0, 2, 3, 1).reshape(Ntot, C)
    y0, stats1 = _prep(x_rows, fc1_w, fc1_b.reshape(1, C))

    eps12 = jnp.stack([eps1, eps2]).reshape(1, 2)
    att, emb = _gnn(
        y0, stats1, bn1_g.reshape(1, C), bn1_b.reshape(1, C), eps12,
        gin1_w1, gin1_b1.reshape(1, C), gin1_w2, gin1_b2.reshape(1, C),
        att_w1[:C], att_w1[C:], att_b1.reshape(1, C),
        att_w2.reshape(1, C), att_b2.reshape(1, 1),
        gin2_w1, gin2_b1.reshape(1, C), gin2_w2, gin2_b2.reshape(1, C))

    t1, stats2 = _ffn1(emb, ffn_w1, ffn_b1.reshape(1, HID))
    v, stats3 = _ffn2(t1, stats2, bn2_g.reshape(1, HID), bn2_b.reshape(1, HID),
                      ffn_w2, ffn_b2.reshape(1, C))
    out_rows = _ffn3(emb, v, stats3, bn3_g.reshape(1, C), bn3_b.reshape(1, C))

    edge_att = att.reshape(Ntot * K, 1)
    out = out_rows.reshape(B, H, W, C).transpose(0, 3, 1, 2)
    return edge_att, out
